# Initial kernel scaffold; baseline (speedup 1.0000x reference)
import jax
import jax.numpy as jnp
from jax.experimental import pallas as pl


def kernel(node_features, positions, sizes, topo_edge_index, topo_edge_attr, params):
    raise NotImplementedError("write your pallas kernel here")


# speedup vs baseline: 4.9570x; 4.9570x over previous
import math

import jax
import jax.numpy as jnp
from jax import lax
from jax.experimental import pallas as pl
from jax.experimental.pallas import tpu as pltpu
from jax.experimental.pallas import tpu_sc as plsc

V = 2048
H = 128
KNN = 8
NLAYERS = 5
ROWBLK = 256
EBLK = 4096
NC, NS = 2, 16
NW = NC * NS
CH = 128
RPS = V // NS

_f32 = jnp.float32


def _relu(x):
    return jnp.maximum(x, 0.0)


def _mm(a, b):
    return jnp.dot(a, b, preferred_element_type=_f32)


def _enc_body(nf, w1, b1, w2, b2, h_o):
    x = _relu(_mm(nf[...], w1[...]) + b1[...])
    h_o[...] = _relu(_mm(x, w2[...]) + b2[...])


def _enc_call(nf_pad, w1, b1, w2, b2):
    return pl.pallas_call(
        _enc_body,
        out_shape=jax.ShapeDtypeStruct((V, H), _f32),
    )(nf_pad, w1, b1, w2, b2)


def _edge_enc_call(e_pad, a1, c1, a2, c2):
    E = e_pad.shape[0]
    blk = pl.BlockSpec((EBLK, H), lambda i: (i, 0))
    full = lambda shape: pl.BlockSpec(shape, lambda i: tuple(0 for _ in shape))
    return pl.pallas_call(
        _enc_body,
        grid=(E // EBLK,),
        in_specs=[blk, full((H, H)), full((1, H)), full((H, H)), full((1, H))],
        out_specs=blk,
        out_shape=jax.ShapeDtypeStruct((E, H), _f32),
    )(e_pad, a1, c1, a2, c2)


def _knn_body(pos, siz, pxc, pyc, swc, shc, idx_o, attr_o):
    i = pl.program_id(0)
    px = pos[:, 0:1]
    py = pos[:, 1:2]
    sw_r = siz[:, 0:1]
    sh_r = siz[:, 1:2]
    dx = px - pxc[...]
    dy = py - pyc[...]
    sq = dx * dx + dy * dy
    row_ids = i * ROWBLK + lax.broadcasted_iota(jnp.int32, (ROWBLK, V), 0)
    col_ids = lax.broadcasted_iota(jnp.int32, (ROWBLK, V), 1)
    eye = row_ids == col_ids
    d = jnp.sqrt(jnp.where(eye, 1.0, sq))
    d = jnp.where(eye, jnp.inf, d)
    swc_b = jnp.broadcast_to(swc[...], (ROWBLK, V))
    shc_b = jnp.broadcast_to(shc[...], (ROWBLK, V))
    idx_cols = []
    attr_cols = []
    zattr = jnp.zeros((ROWBLK, 10), _f32)
    for _ in range(KNN):
        mn = jnp.min(d, axis=1, keepdims=True)
        ismin = d == mn
        idx = jnp.min(jnp.where(ismin, col_ids, V), axis=1, keepdims=True)
        onehot = col_ids == idx
        dxj = jnp.sum(jnp.where(onehot, dx, 0.0), axis=1, keepdims=True)
        dyj = jnp.sum(jnp.where(onehot, dy, 0.0), axis=1, keepdims=True)
        swd = jnp.sum(jnp.where(onehot, swc_b, 0.0), axis=1, keepdims=True)
        shd = jnp.sum(jnp.where(onehot, shc_b, 0.0), axis=1, keepdims=True)
        gapx = _relu(jnp.abs(dxj) - sw_r * 0.5 - swd * 0.5)
        gapy = _relu(jnp.abs(dyj) - sh_r * 0.5 - shd * 0.5)
        ratio = (swd * shd) / (sw_r * sh_r + 1e-8)
        attr_cols.append(jnp.concatenate(
            [dxj, dyj, mn, gapx, gapy, ratio, zattr], axis=1))
        idx_cols.append(idx)
        d = jnp.where(onehot, jnp.inf, d)
    idx_o[...] = jnp.concatenate(
        idx_cols + [jnp.zeros((ROWBLK, H - KNN), jnp.int32)], axis=1)
    attr_o[...] = jnp.concatenate(attr_cols, axis=1)


def _knn_call(pos_pad, siz_pad, pxc, pyc, swc, shc):
    grid = V // ROWBLK
    blk = pl.BlockSpec((ROWBLK, H), lambda i: (i, 0))
    row = pl.BlockSpec((1, V), lambda i: (0, 0))
    return pl.pallas_call(
        _knn_body,
        grid=(grid,),
        in_specs=[blk, blk, row, row, row, row],
        out_specs=(blk, blk),
        out_shape=(
            jax.ShapeDtypeStruct((V, H), jnp.int32),
            jax.ShapeDtypeStruct((V, H), _f32),
        ),
    )(pos_pad, siz_pad, pxc, pyc, swc, shc)


def _sc_gather(h, st, dt, ss, ds_):
    ET = st.shape[0]
    ES = ss.shape[0]
    mesh = plsc.VectorSubcoreMesh(
        core_axis_name="c", subcore_axis_name="s",
        num_cores=NC, num_subcores=NS)

    def body(h_h, st_h, dt_h, ss_h, ds_h, o_st, o_dt, o_ss, o_ds,
             idx_v, buf, sem):
        c = lax.axis_index("c")
        s = lax.axis_index("s")
        wid = c * NS + s

        def gath(idx_h, out_h, epw):
            for ci in range(epw // CH):
                base = wid * epw + ci * CH
                pltpu.sync_copy(idx_h.at[pl.ds(base, CH)], idx_v)
                pltpu.async_copy(h_h.at[idx_v], buf, sem).wait()
                pltpu.sync_copy(buf, out_h.at[pl.ds(base, CH)])

        gath(st_h, o_st, ET // NW)
        gath(dt_h, o_dt, ET // NW)
        gath(ss_h, o_ss, ES // NW)
        gath(ds_h, o_ds, ES // NW)

    f = pl.kernel(
        body,
        out_type=(
            jax.ShapeDtypeStruct((ET, H), _f32),
            jax.ShapeDtypeStruct((ET, H), _f32),
            jax.ShapeDtypeStruct((ES, H), _f32),
            jax.ShapeDtypeStruct((ES, H), _f32),
        ),
        mesh=mesh,
        scratch_types=[
            pltpu.VMEM((CH,), jnp.int32),
            pltpu.VMEM((CH, H), _f32),
            pltpu.SemaphoreType.DMA,
        ],
    )
    return f(h, st, dt, ss, ds_)


def _msg_body(hs, hd, ee, w, bb, o):
    m = jnp.concatenate([hs[...], hd[...], ee[...]], axis=1)
    o[...] = _relu(_mm(m, w[...]) + bb[...])


def _msg_call(hs, hd, ee, w, bb):
    E = hs.shape[0]
    blk = pl.BlockSpec((EBLK, H), lambda i: (i, 0))
    full = lambda shape: pl.BlockSpec(shape, lambda i: tuple(0 for _ in shape))
    return pl.pallas_call(
        _msg_body,
        grid=(E // EBLK,),
        in_specs=[blk, blk, blk, full((3 * H, H)), full((1, H))],
        out_specs=blk,
        out_shape=jax.ShapeDtypeStruct((E, H), _f32),
    )(hs, hd, ee, w, bb)


def _sc_scatter(mt, dtt, ms, dts, zz):
    ET = mt.shape[0]
    ES = ms.shape[0]
    mesh = plsc.VectorSubcoreMesh(
        core_axis_name="c", subcore_axis_name="s",
        num_cores=NC, num_subcores=NS)

    def body(mt_h, dtt_h, ms_h, dts_h, zz_h, out, ib, mv, acct, accs, sem):
        c = lax.axis_index("c")
        s = lax.axis_index("s")
        wid = c * NS + s
        r0 = s * RPS
        pltpu.sync_copy(zz_h.at[pl.ds(r0, RPS)], acct.at[pl.ds(r0, RPS)])
        pltpu.sync_copy(zz_h.at[pl.ds(r0, RPS)], accs.at[pl.ds(r0, RPS)])
        plsc.subcore_barrier()

        def scat(m_h, dst_h, acc, epw):
            for ci in range(epw // CH):
                base = wid * epw + ci * CH
                pltpu.sync_copy(dst_h.at[pl.ds(base, CH)], ib)
                pltpu.sync_copy(m_h.at[pl.ds(base, CH)], mv)
                pltpu.sync_copy(mv, acc.at[ib], add=True)

        scat(mt_h, dtt_h, acct, ET // NW)
        scat(ms_h, dts_h, accs, ES // NW)
        plsc.subcore_barrier()
        pltpu.sync_copy(acct.at[pl.ds(r0, RPS)], out.at[0, c, pl.ds(r0, RPS)])
        pltpu.sync_copy(accs.at[pl.ds(r0, RPS)], out.at[1, c, pl.ds(r0, RPS)])

    f = pl.kernel(
        body,
        out_type=jax.ShapeDtypeStruct((2, NC, V, H), _f32),
        mesh=mesh,
        scratch_types=[
            pltpu.VMEM((CH,), jnp.int32),
            pltpu.VMEM((CH, H), _f32),
            pltpu.VMEM_SHARED((V, H), _f32),
            pltpu.VMEM_SHARED((V, H), _f32),
            pltpu.SemaphoreType.DMA,
        ],
    )
    return f(mt, dtt, ms, dts, zz)


def _upd_body(h, agg, wut, but, wus, bus, wf, bf, g, bl, hn_o):
    hh = h[...]
    at = agg[0, 0] + agg[0, 1]
    asx = agg[1, 0] + agg[1, 1]
    ht = _relu(_mm(jnp.concatenate([hh, at], axis=1), wut[...]) + but[...])
    hs = _relu(_mm(jnp.concatenate([hh, asx], axis=1), wus[...]) + bus[...])
    f = _mm(jnp.concatenate([ht, hs], axis=1), wf[...]) + bf[...] + ht + hs
    mu = jnp.mean(f, axis=-1, keepdims=True)
    var = jnp.mean((f - mu) * (f - mu), axis=-1, keepdims=True)
    hn_o[...] = (f - mu) / jnp.sqrt(var + 1e-5) * g[...] + bl[...]


def _upd_call(h, agg, wut, but, wus, bus, wf, bf, g, bl):
    return pl.pallas_call(
        _upd_body,
        out_shape=jax.ShapeDtypeStruct((V, H), _f32),
    )(h, agg, wut, but, wus, bus, wf, bf, g, bl)


def _heads_body(h, wd1, bd1, wd2, bd2, wh1, bh1, wh2, bh2,
                wv1, bv1, wv2, bv2, wv3, bv3, disp_o, heat_o, val_o):
    hh = h[...]
    x = _relu(_mm(hh, wd1[...]) + bd1[...])
    disp_o[...] = _mm(x, wd2[...]) + bd2[...]
    y = _relu(_mm(hh, wh1[...]) + bh1[...])
    heat_o[...] = _mm(y, wh2[...]) + bh2[...]
    ge = jnp.sum(hh, axis=0, keepdims=True) * (1.0 / math.sqrt(V))
    v = _relu(_mm(ge, wv1[...]) + bv1[...])
    v = _relu(_mm(v, wv2[...]) + bv2[...])
    val_o[...] = _mm(v, wv3[...]) + bv3[...]


def _heads_call(h, wd1, bd1, wd2, bd2, wh1, bh1, wh2, bh2,
                wv1, bv1, wv2, bv2, wv3, bv3):
    return pl.pallas_call(
        _heads_body,
        out_shape=(
            jax.ShapeDtypeStruct((V, H), _f32),
            jax.ShapeDtypeStruct((V, 32 * 32), _f32),
            jax.ShapeDtypeStruct((1, H), _f32),
        ),
    )(h, wd1, bd1, wd2, bd2, wh1, bh1, wh2, bh2,
      wv1, bv1, wv2, bv2, wv3, bv3)


def _pad_cols(a, n):
    return jnp.pad(a, ((0, 0), (0, n - a.shape[1])))


def _pad_rows(a, n):
    return jnp.pad(a, ((0, n - a.shape[0]), (0, 0)))


def _bias(b, n=H):
    return jnp.pad(b, (0, n - b.shape[0]))[None, :]


def kernel(node_features, positions, sizes, topo_edge_index, topo_edge_attr, params):
    p = params

    (ew1, eb1), (ew2, eb2) = p['node_enc']
    (ta1, tc1), (ta2, tc2) = p['topo_edge_enc']
    (sa1, sc1), (sa2, sc2) = p['spatial_edge_enc']

    wm_t = [p['topo_layers'][l]['msg'][0] for l in range(NLAYERS)]
    bm_t = [p['topo_layers'][l]['msg'][1][None, :] for l in range(NLAYERS)]
    wm_s = [p['spatial_layers'][l]['msg'][0] for l in range(NLAYERS)]
    bm_s = [p['spatial_layers'][l]['msg'][1][None, :] for l in range(NLAYERS)]
    wu_t = [p['topo_layers'][l]['upd'][0] for l in range(NLAYERS)]
    bu_t = [p['topo_layers'][l]['upd'][1][None, :] for l in range(NLAYERS)]
    wu_s = [p['spatial_layers'][l]['upd'][0] for l in range(NLAYERS)]
    bu_s = [p['spatial_layers'][l]['upd'][1][None, :] for l in range(NLAYERS)]
    wf = [p['fusion'][l][0] for l in range(NLAYERS)]
    bf = [p['fusion'][l][1][None, :] for l in range(NLAYERS)]
    g_ln = [p['ln'][l][0][None, :] for l in range(NLAYERS)]
    b_ln = [p['ln'][l][1][None, :] for l in range(NLAYERS)]

    wd1, bd1 = p['disp_head']['mlp'][0]
    wd2, bd2 = p['disp_head']['out']
    wh1, bh1 = p['heat_head']['mlp'][0]
    wh2, bh2 = p['heat_head']['out']
    (wv1, bv1), (wv2, bv2), (wv3, bv3) = p['value']

    nf_pad = _pad_cols(node_features, H)
    ea_pad = _pad_cols(topo_edge_attr, H)
    pos_pad = _pad_cols(positions, H)
    siz_pad = _pad_cols(sizes, H)
    pxc = positions[:, 0][None, :]
    pyc = positions[:, 1][None, :]
    swc = sizes[:, 0][None, :]
    shc = sizes[:, 1][None, :]
    src_t = topo_edge_index[0].astype(jnp.int32)
    dst_t = topo_edge_index[1].astype(jnp.int32)
    src_s = jnp.repeat(jnp.arange(V, dtype=jnp.int32), KNN)
    zz = jnp.zeros((V, H), _f32)

    h = _enc_call(nf_pad, _pad_rows(ew1, H), eb1[None, :], ew2, eb2[None, :])
    topo_e = _edge_enc_call(ea_pad, _pad_rows(ta1, H), tc1[None, :],
                            ta2, tc2[None, :])
    idx_pack, attr_pack = _knn_call(pos_pad, siz_pad, pxc, pyc, swc, shc)
    dst_s = idx_pack[:, :KNN].reshape(-1)
    sp_in = _pad_cols(attr_pack.reshape(V * KNN, 16), H)
    sp_e = _edge_enc_call(sp_in, _pad_rows(sa1, H), sc1[None, :],
                          sa2, sc2[None, :])

    for l in range(NLAYERS):
        hst, hdt, hss, hds = _sc_gather(h, src_t, dst_t, src_s, dst_s)
        mt = _msg_call(hst, hdt, topo_e, wm_t[l], bm_t[l])
        ms = _msg_call(hss, hds, sp_e, wm_s[l], bm_s[l])
        agg = _sc_scatter(mt, dst_t, ms, dst_s, zz)
        h = _upd_call(h, agg, wu_t[l], bu_t[l], wu_s[l], bu_s[l],
                      wf[l], bf[l], g_ln[l], b_ln[l])

    disp_pad, heat, val = _heads_call(
        h, wd1, bd1[None, :], _pad_cols(wd2, H), _bias(bd2),
        wh1, bh1[None, :], wh2, bh2[None, :],
        _pad_cols(wv1, H), _bias(bv1),
        _pad_cols(_pad_rows(wv2, H), H), _bias(bv2),
        _pad_cols(_pad_rows(wv3, H), H), _bias(bv3))

    return (disp_pad[:, :2], heat, val[:, 0], h)
